# Initial kernel scaffold; baseline (speedup 1.0000x reference)
import jax
import jax.numpy as jnp
from jax.experimental import pallas as pl


def kernel(x, weight, bias):
    raise NotImplementedError("write your pallas kernel here")


# speedup vs baseline: 1.6075x; 1.6075x over previous
import jax
import jax.numpy as jnp
from jax.experimental import pallas as pl
from jax.experimental.pallas import tpu as pltpu


def _expand_kernel(x_ref, w_ref, b_ref, o_ref):
    acc = jnp.dot(x_ref[...], w_ref[...], preferred_element_type=jnp.float32)
    o_ref[...] = (acc + b_ref[...].astype(jnp.float32)).astype(o_ref.dtype)


def kernel(x, weight, bias):
    B, Cin = x.shape
    F = weight.shape[1]
    out_channels, Y, X = 128, 8, 8

    TN = next((t for t in (512, 256, 128) if F % t == 0), F)
    num_j = F // TN

    out_flat = pl.pallas_call(
        _expand_kernel,
        out_shape=jax.ShapeDtypeStruct((B, F), x.dtype),
        grid=(num_j,),
        in_specs=[
            pl.BlockSpec((B, Cin), lambda j: (0, 0)),
            pl.BlockSpec((Cin, TN), lambda j: (0, j)),
            pl.BlockSpec((1, TN), lambda j: (0, j)),
        ],
        out_specs=pl.BlockSpec((B, TN), lambda j: (0, j)),
        compiler_params=pltpu.CompilerParams(
            dimension_semantics=("parallel",)),
        cost_estimate=pl.CostEstimate(
            flops=2 * B * Cin * F,
            transcendentals=0,
            bytes_accessed=(B * Cin + Cin * F + B * F) * 4,
        ),
    )(x, weight, bias.reshape(1, F))

    return out_flat.reshape(B, out_channels, Y, X)
